# Initial kernel scaffold; baseline (speedup 1.0000x reference)
import jax
import jax.numpy as jnp
from jax.experimental import pallas as pl


def kernel(event_type, fault_class, syscall_class, opcode_family, transition_type, result_class, W_event, W_fault, W_syscall, W_opcode, W_trans, W_result, gate_W, gate_b, trans_W, trans_b, ln_gamma, ln_beta):
    raise NotImplementedError("write your pallas kernel here")


# speedup vs baseline: 2.4699x; 2.4699x over previous
import functools
import jax
import jax.numpy as jnp
from jax import lax
from jax.experimental import pallas as pl
from jax.experimental.pallas import tpu as pltpu
from jax.experimental.pallas import tpu_sc as plsc

DW = 8
NW = 32
CHUNK = 2400
BLK = 2048


def _gather_kernel(n_total, m_per_w):
    mesh = plsc.VectorSubcoreMesh(core_axis_name="c", subcore_axis_name="s")

    @functools.partial(
        pl.kernel,
        mesh=mesh,
        out_type=jax.ShapeDtypeStruct((n_total, DW), jnp.float32),
        scratch_types=[
            pltpu.VMEM((CHUNK,), jnp.int32),
            pltpu.VMEM((CHUNK, DW), jnp.float32),
            pltpu.SemaphoreType.DMA,
        ],
        compiler_params=pltpu.CompilerParams(use_tc_tiling_on_sc=False),
    )
    def gather_k(table_hbm, idx_hbm, out_hbm, idx_v, rows_v, sem):
        wid = lax.axis_index("s") * 2 + lax.axis_index("c")
        base = wid * m_per_w

        def step(i, carry):
            off = base + i * CHUNK
            pltpu.sync_copy(idx_hbm.at[pl.ds(off, CHUNK)], idx_v)
            pltpu.async_copy(table_hbm.at[idx_v], rows_v, sem).wait()
            pltpu.sync_copy(rows_v, out_hbm.at[pl.ds(off, CHUNK)])
            return carry

        lax.fori_loop(0, m_per_w // CHUNK, step, 0)

    return gather_k


def _fuse_body(x_ref, wg_ref, wt_ref, gb_ref, tb_ref, gam_ref, bet_ref, o_ref):
    x = x_ref[...]
    wg = wg_ref[...]
    wt = wt_ref[...]
    g_lin = gb_ref[...]
    t_lin = tb_ref[...]
    for t in range(6):
        g_lin = g_lin + jnp.dot(x[t], wg[t], preferred_element_type=jnp.float32)
        t_lin = t_lin + jnp.dot(x[t], wt[t], preferred_element_type=jnp.float32)
    gate = jax.nn.sigmoid(g_lin * 1.2)
    z = gate * t_lin
    mu = jnp.mean(z, axis=-1, keepdims=True)
    zc = z - mu
    var = jnp.mean(zc * zc, axis=-1, keepdims=True)
    z_norm = zc * lax.rsqrt(var + 1e-5)
    o_ref[...] = z_norm * gam_ref[...] + bet_ref[...]


def kernel(event_type, fault_class, syscall_class, opcode_family,
           transition_type, result_class,
           W_event, W_fault, W_syscall, W_opcode, W_trans, W_result,
           gate_W, gate_b, trans_W, trans_b, ln_gamma, ln_beta):
    tables = [W_event, W_fault, W_syscall, W_opcode, W_trans, W_result]
    idxs = [event_type, fault_class, syscall_class, opcode_family,
            transition_type, result_class]
    widths = [t.shape[1] for t in tables]

    n = event_type.size
    n_total = 6 * n

    padded, shifted, row_base = [], [], 0
    for W, ix in zip(tables, idxs):
        padded.append(jnp.pad(W, ((0, 0), (0, DW - W.shape[1]))))
        shifted.append(ix.reshape(-1).astype(jnp.int32) + row_base)
        row_base += W.shape[0]
    big_table = jnp.concatenate(padded, axis=0)
    idx_all = jnp.concatenate(shifted)

    m_per_w = n_total // NW
    gathered = _gather_kernel(n_total, m_per_w)(big_table, idx_all)
    gathered = gathered.reshape(6, n, DW)

    def seg_w(W):
        out, r = [], 0
        for w in widths:
            out.append(jnp.pad(W[r:r + w], ((0, DW - w), (0, 0))))
            r += w
        return jnp.stack(out)

    wg = seg_w(gate_W)
    wt = seg_w(trans_W)

    out = pl.pallas_call(
        _fuse_body,
        grid=(n // BLK,),
        in_specs=[
            pl.BlockSpec((6, BLK, DW), lambda i: (0, i, 0)),
            pl.BlockSpec((6, DW, 128), lambda i: (0, 0, 0)),
            pl.BlockSpec((6, DW, 128), lambda i: (0, 0, 0)),
            pl.BlockSpec((1, 128), lambda i: (0, 0)),
            pl.BlockSpec((1, 128), lambda i: (0, 0)),
            pl.BlockSpec((1, 128), lambda i: (0, 0)),
            pl.BlockSpec((1, 128), lambda i: (0, 0)),
        ],
        out_specs=pl.BlockSpec((BLK, 128), lambda i: (i, 0)),
        out_shape=jax.ShapeDtypeStruct((n, 128), jnp.float32),
    )(gathered, wg, wt,
      gate_b.reshape(1, 128), trans_b.reshape(1, 128),
      ln_gamma.reshape(1, 128), ln_beta.reshape(1, 128))

    return out.reshape(event_type.shape + (128,))


# speedup vs baseline: 2.8298x; 1.1457x over previous
import functools
import jax
import jax.numpy as jnp
from jax import lax
from jax.experimental import pallas as pl
from jax.experimental.pallas import tpu as pltpu
from jax.experimental.pallas import tpu_sc as plsc

DW = 8
PK = 128 // DW
NW = 32
NT = 6


def _gather_kernel(n, chunk):
    mesh = plsc.VectorSubcoreMesh(core_axis_name="c", subcore_axis_name="s")
    tpw = n // NW
    g_steps = tpw // chunk
    pc = chunk // PK
    lines_t = n // PK

    @functools.partial(
        pl.kernel,
        mesh=mesh,
        out_type=jax.ShapeDtypeStruct((NT * lines_t, 128), jnp.float32),
        scratch_types=[
            pltpu.VMEM((2, chunk), jnp.int32),
            pltpu.VMEM((2, PK, pc, DW), jnp.float32),
            pltpu.SemaphoreType.DMA,
            pltpu.SemaphoreType.DMA,
            pltpu.SemaphoreType.DMA,
            pltpu.SemaphoreType.DMA,
        ],
        compiler_params=pltpu.CompilerParams(use_tc_tiling_on_sc=False),
    )
    def gather_k(table_hbm, idx_hbm, out_hbm, idx_v, rows_v, g0, g1, w0, w1):
        wid = lax.axis_index("s") * 2 + lax.axis_index("c")
        gsem = [g0, g1]
        wsem = [w0, w1]
        chunks = [(t, g) for t in range(NT) for g in range(g_steps)]

        def load_idx(c, s):
            t, g = chunks[c]
            off = t * n + wid * tpw + g * chunk
            pltpu.sync_copy(idx_hbm.at[pl.ds(off, chunk)], idx_v.at[s])

        def start_gathers(s):
            return [
                pltpu.async_copy(
                    table_hbm.at[idx_v.at[s, pl.ds(k * pc, pc)]],
                    rows_v.at[s, k],
                    gsem[s])
                for k in range(PK)
            ]

        def start_wb(c, s):
            t, g = chunks[c]
            line0 = t * lines_t + wid * (tpw // PK) + g * pc
            return [
                pltpu.async_copy(
                    rows_v.at[s, k],
                    out_hbm.at[pl.ds(line0, pc), pl.ds(k * DW, DW)],
                    wsem[s])
                for k in range(PK)
            ]

        n_chunks = len(chunks)
        load_idx(0, 0)
        g_h = [start_gathers(0), None]
        w_h = [None, None]
        for c in range(n_chunks):
            s = c & 1
            o = s ^ 1
            if c + 1 < n_chunks:
                if w_h[o] is not None:
                    for h in w_h[o]:
                        h.wait()
                load_idx(c + 1, o)
                g_h[o] = start_gathers(o)
            for h in g_h[s]:
                h.wait()
            w_h[s] = start_wb(c, s)
        for hs in w_h:
            if hs is not None:
                for h in hs:
                    h.wait()

    return gather_k


def _make_fuse_body(pc):
    def _fuse_body(x_ref, w2_ref, b2_ref, gam_ref, bet_ref, o_ref):
        w2 = w2_ref[...]
        b2 = b2_ref[...]
        for k in range(PK):
            xk = jnp.concatenate(
                [x_ref[t][:, k * DW:(k + 1) * DW] for t in range(NT)],
                axis=1)
            lin = jnp.dot(xk, w2, preferred_element_type=jnp.float32) + b2
            gate = jax.nn.sigmoid(lin[:, :128] * 1.2)
            z = gate * lin[:, 128:]
            mu = jnp.mean(z, axis=-1, keepdims=True)
            zc = z - mu
            var = jnp.mean(zc * zc, axis=-1, keepdims=True)
            z_norm = zc * lax.rsqrt(var + 1e-5)
            o_ref[pl.ds(k * pc, pc), :] = z_norm * gam_ref[...] + bet_ref[...]
    return _fuse_body


def kernel(event_type, fault_class, syscall_class, opcode_family,
           transition_type, result_class,
           W_event, W_fault, W_syscall, W_opcode, W_trans, W_result,
           gate_W, gate_b, trans_W, trans_b, ln_gamma, ln_beta):
    tables = [W_event, W_fault, W_syscall, W_opcode, W_trans, W_result]
    idxs = [event_type, fault_class, syscall_class, opcode_family,
            transition_type, result_class]
    widths = [t.shape[1] for t in tables]

    n = event_type.size
    chunk = n // NW
    pc = chunk // PK

    padded, shifted, row_base = [], [], 0
    for W, ix in zip(tables, idxs):
        padded.append(jnp.pad(W, ((0, 0), (0, DW - W.shape[1]))))
        shifted.append(ix.reshape(-1).astype(jnp.int32) + row_base)
        row_base += W.shape[0]
    big_table = jnp.concatenate(padded, axis=0)
    idx_all = jnp.concatenate(shifted)

    gathered = _gather_kernel(n, chunk)(big_table, idx_all)
    gathered = gathered.reshape(NT, n // PK, 128)

    def seg_w(W):
        out, r = [], 0
        for w in widths:
            out.append(jnp.pad(W[r:r + w], ((0, DW - w), (0, 0))))
            r += w
        return jnp.concatenate(out, axis=0)

    w2 = jnp.concatenate([seg_w(gate_W), seg_w(trans_W)], axis=1)
    b2 = jnp.concatenate([gate_b, trans_b]).reshape(1, 256)

    out = pl.pallas_call(
        _make_fuse_body(pc),
        grid=(n // chunk,),
        in_specs=[
            pl.BlockSpec((NT, pc, 128), lambda i: (0, i, 0)),
            pl.BlockSpec((NT * DW, 256), lambda i: (0, 0)),
            pl.BlockSpec((1, 256), lambda i: (0, 0)),
            pl.BlockSpec((1, 128), lambda i: (0, 0)),
            pl.BlockSpec((1, 128), lambda i: (0, 0)),
        ],
        out_specs=pl.BlockSpec((chunk, 128), lambda i: (i, 0)),
        out_shape=jax.ShapeDtypeStruct((n, 128), jnp.float32),
    )(gathered, w2, b2, ln_gamma.reshape(1, 128), ln_beta.reshape(1, 128))

    return out.reshape(event_type.shape + (128,))
